# baseline (speedup 1.0000x reference)
import functools

import jax
import jax.numpy as jnp
from jax import lax
from jax.experimental import pallas as pl
from jax.experimental.pallas import tpu as pltpu
from jax.experimental.pallas import tpu_sc as plsc

N = 10000
E = 320000
D_IN = 128
D_H = 128
D_OUT = 64
EPSV = 1e-5

C = 128
NW = 32
CHUNKS_PER_W = 80
E_PAD = NW * CHUNKS_PER_W * C
R = 10240
ROWS_PER_SUB = R // 16
K_CORE0 = 160
K_CORE1 = 0


_SC_MESH = plsc.VectorSubcoreMesh(core_axis_name="c", subcore_axis_name="s")


def _make_sc_agg(D):
    MMAX = 40
    scratch = [
        pltpu.VMEM((MMAX, C), jnp.int32),
        pltpu.VMEM((MMAX, C), jnp.int32),
        pltpu.VMEM((C, D), jnp.float32),
        pltpu.VMEM((C, D), jnp.float32),
        pltpu.VMEM_SHARED((R, D), jnp.float32),
        pltpu.SemaphoreType.DMA,
        pltpu.SemaphoreType.DMA,
    ]

    def body(t_hbm, srcp, dstp, zrows, out_hbm,
             src_v, dst_v, rows_a, rows_b, acc_sh, sem_a, sem_b):
        c = lax.axis_index("c")
        s = lax.axis_index("s")
        base_r = s * ROWS_PER_SUB

        @pl.loop(0, ROWS_PER_SUB // 128)
        def _(i):
            pltpu.sync_copy(zrows, acc_sh.at[pl.ds(base_r + i * 128, 128)])

        plsc.subcore_barrier()

        def g_start(j, buf, sem):
            pltpu.async_copy(t_hbm.at[src_v.at[j]], buf, sem)

        def g_wait(buf, sem):
            pltpu.make_async_copy(t_hbm.at[src_v.at[0]], buf, sem).wait()

        def run_agg(base_chunk, K):
            HF = min(K // 2, MMAX) if K else 0
            if K == 0:
                return

            @pl.loop(0, K // HF)
            def _(h):
                base = base_chunk + h * HF
                pltpu.sync_copy(srcp.at[pl.ds(base, HF)],
                                src_v.at[pl.ds(0, HF)])
                pltpu.sync_copy(dstp.at[pl.ds(base, HF)],
                                dst_v.at[pl.ds(0, HF)])

                g_start(0, rows_a, sem_a)

                @pl.loop(0, HF // 2)
                def _(i):
                    a = 2 * i
                    g_start(a + 1, rows_b, sem_b)
                    g_wait(rows_a, sem_a)
                    pltpu.sync_copy(rows_a, acc_sh.at[dst_v.at[a]], add=True)

                    @pl.when(i < HF // 2 - 1)
                    def _():
                        g_start(a + 2, rows_a, sem_a)
                    g_wait(rows_b, sem_b)
                    pltpu.sync_copy(rows_b, acc_sh.at[dst_v.at[a + 1]],
                                    add=True)

        @pl.when(c == 0)
        def _():
            run_agg(s * K_CORE0, K_CORE0)

        if K_CORE1 > 0:
            @pl.when(c == 1)
            def _():
                run_agg(16 * K_CORE0 + s * K_CORE1, K_CORE1)

        plsc.subcore_barrier()

        pltpu.sync_copy(acc_sh.at[pl.ds(base_r, ROWS_PER_SUB)],
                        out_hbm.at[c, pl.ds(base_r, ROWS_PER_SUB)])

    return pl.kernel(body, out_type=jax.ShapeDtypeStruct((2, R, D), jnp.float32),
                     mesh=_SC_MESH, scratch_types=scratch)


def _make_sc_deg():
    scratch = [
        pltpu.VMEM((CHUNKS_PER_W, C), jnp.int32),
        pltpu.VMEM((C, D_H), jnp.float32),
        pltpu.VMEM_SHARED((R, D_H), jnp.float32),
    ]

    def body(dstp, zrows, ones_h, deg_hbm, dst_v, ones_v, deg_sh):
        c = lax.axis_index("c")
        s = lax.axis_index("s")
        w = s * 2 + c
        base_r = s * ROWS_PER_SUB

        pltpu.sync_copy(dstp.at[pl.ds(w * CHUNKS_PER_W, CHUNKS_PER_W)], dst_v)

        @pl.loop(0, ROWS_PER_SUB // 128)
        def _(i):
            pltpu.sync_copy(zrows, deg_sh.at[pl.ds(base_r + i * 128, 128)])
        pltpu.sync_copy(ones_h, ones_v)

        plsc.subcore_barrier()

        @pl.loop(0, CHUNKS_PER_W)
        def _(k):
            pltpu.sync_copy(ones_v, deg_sh.at[dst_v.at[k]], add=True)

        plsc.subcore_barrier()

        pltpu.sync_copy(deg_sh.at[pl.ds(base_r, ROWS_PER_SUB)],
                        deg_hbm.at[c, pl.ds(base_r, ROWS_PER_SUB)])

    return pl.kernel(
        body, out_type=jax.ShapeDtypeStruct((2, R, D_H), jnp.float32),
        mesh=_SC_MESH, scratch_types=scratch)


_sc_agg_128 = _make_sc_agg(D_H)
_sc_deg = _make_sc_deg()


def _tc_pre(x_ref, w_ref, o_ref):
    o_ref[...] = jnp.dot(x_ref[...], w_ref[...],
                         preferred_element_type=jnp.float32)


def _tc_mid(aggp, degp, h_prev, Wr, b, g, be, Wln, h_out, t_out):
    deg = jnp.maximum(degp[0] + degp[1], 1.0)
    agg = aggp[0] + aggp[1]
    mean = agg / deg[:, None]
    z = mean + jnp.dot(h_prev[...], Wr[...],
                       preferred_element_type=jnp.float32) + b[...]
    m = jnp.mean(z, axis=0)
    v = jnp.mean((z - m) ** 2, axis=0)
    h = jnp.maximum((z - m) / jnp.sqrt(v + EPSV) * g[...] + be[...], 0.0)
    h_out[...] = h
    t_out[...] = jnp.dot(h, Wln[...], preferred_element_type=jnp.float32)


def _tc_fin(aggp, degp, h_prev, Wr, b, o_ref):
    deg = jnp.maximum(degp[0] + degp[1], 1.0)
    agg = aggp[0] + aggp[1]
    o_ref[...] = agg / deg[:, None] + jnp.dot(
        h_prev[...], Wr[...], preferred_element_type=jnp.float32) + b[...]


def kernel(x, edge_index, Wl1, Wr1, b1, g1, be1, Wl2, Wr2, b2, g2, be2,
           Wl3, Wr3, b3):
    src = edge_index[0]
    dst = edge_index[1]
    pad = E_PAD - E
    srcp = jnp.concatenate([src, jnp.zeros((pad,), jnp.int32)])
    srcp = srcp.reshape(E_PAD // C, C)
    dstp = jnp.concatenate([dst, jnp.full((pad,), N, jnp.int32)])
    dstp = dstp.reshape(E_PAD // C, C)
    z128 = jnp.zeros((128, D_H), jnp.float32)
    ones128 = jnp.ones((C, D_H), jnp.float32)

    f32 = jnp.float32
    t1 = pl.pallas_call(
        _tc_pre, out_shape=jax.ShapeDtypeStruct((N, D_H), f32))(x, Wl1)

    degp = _sc_deg(dstp, z128, ones128)
    degv = degp[:, :N, 0]
    agg1p = _sc_agg_128(t1, srcp, dstp, z128)

    h1, t2 = pl.pallas_call(
        _tc_mid,
        out_shape=[jax.ShapeDtypeStruct((N, D_H), f32),
                   jax.ShapeDtypeStruct((N, D_H), f32)],
    )(agg1p[:, :N], degv, x, Wr1, b1, g1, be1, Wl2)

    agg2p = _sc_agg_128(t2, srcp, dstp, z128)

    Wl3p = jnp.pad(Wl3, ((0, 0), (0, D_H - D_OUT)))
    h2, t3 = pl.pallas_call(
        _tc_mid,
        out_shape=[jax.ShapeDtypeStruct((N, D_H), f32),
                   jax.ShapeDtypeStruct((N, D_H), f32)],
    )(agg2p[:, :N], degv, h1, Wr2, b2, g2, be2, Wl3p)

    agg3p = _sc_agg_128(t3, srcp, dstp, z128)

    out = pl.pallas_call(
        _tc_fin, out_shape=jax.ShapeDtypeStruct((N, D_OUT), f32),
    )(agg3p[:, :N, :D_OUT], degv, h2, Wr3, b3)
    return out


# speedup vs baseline: 1.2528x; 1.2528x over previous
import functools

import jax
import jax.numpy as jnp
from jax import lax
from jax.experimental import pallas as pl
from jax.experimental.pallas import tpu as pltpu
from jax.experimental.pallas import tpu_sc as plsc

N = 10000
E = 320000
D_IN = 128
D_H = 128
D_OUT = 64
EPSV = 1e-5

C = 128
NW = 32
CHUNKS_PER_W = 80
E_PAD = NW * CHUNKS_PER_W * C
R = 10240
ROWS_PER_SUB = R // 16
K_CORE0 = 112
K_CORE1 = 48


_SC_MESH = plsc.VectorSubcoreMesh(core_axis_name="c", subcore_axis_name="s")


def _make_sc_agg(D, tc_tiling=True):
    MMAX = 40
    scratch = [
        pltpu.VMEM((MMAX, C), jnp.int32),
        pltpu.VMEM((MMAX, C), jnp.int32),
        pltpu.VMEM((C, D), jnp.float32),
        pltpu.VMEM((C, D), jnp.float32),
        pltpu.VMEM_SHARED((R, D), jnp.float32),
        pltpu.SemaphoreType.DMA,
        pltpu.SemaphoreType.DMA,
    ]

    def body(t_hbm, srcp, dstp, zrows, out_hbm,
             src_v, dst_v, rows_a, rows_b, acc_sh, sem_a, sem_b):
        c = lax.axis_index("c")
        s = lax.axis_index("s")
        base_r = s * ROWS_PER_SUB

        @pl.loop(0, ROWS_PER_SUB // 128)
        def _(i):
            pltpu.sync_copy(zrows, acc_sh.at[pl.ds(base_r + i * 128, 128)])

        plsc.subcore_barrier()

        def g_start(j, buf, sem):
            pltpu.async_copy(t_hbm.at[src_v.at[j]], buf, sem)

        def g_wait(buf, sem):
            pltpu.make_async_copy(t_hbm.at[src_v.at[0]], buf, sem).wait()

        def run_agg(base_chunk, K):
            HF = min(K // 2, MMAX) if K else 0
            if K == 0:
                return

            @pl.loop(0, K // HF)
            def _(h):
                base = base_chunk + h * HF
                pltpu.sync_copy(srcp.at[pl.ds(base, HF)],
                                src_v.at[pl.ds(0, HF)])
                pltpu.sync_copy(dstp.at[pl.ds(base, HF)],
                                dst_v.at[pl.ds(0, HF)])

                g_start(0, rows_a, sem_a)

                @pl.loop(0, HF // 2)
                def _(i):
                    a = 2 * i
                    g_start(a + 1, rows_b, sem_b)
                    g_wait(rows_a, sem_a)
                    pltpu.sync_copy(rows_a, acc_sh.at[dst_v.at[a]], add=True)

                    @pl.when(i < HF // 2 - 1)
                    def _():
                        g_start(a + 2, rows_a, sem_a)
                    g_wait(rows_b, sem_b)
                    pltpu.sync_copy(rows_b, acc_sh.at[dst_v.at[a + 1]],
                                    add=True)

        @pl.when(c == 0)
        def _():
            run_agg(s * K_CORE0, K_CORE0)

        if K_CORE1 > 0:
            @pl.when(c == 1)
            def _():
                run_agg(16 * K_CORE0 + s * K_CORE1, K_CORE1)

        plsc.subcore_barrier()

        pltpu.sync_copy(acc_sh.at[pl.ds(base_r, ROWS_PER_SUB)],
                        out_hbm.at[c, pl.ds(base_r, ROWS_PER_SUB)])

    return pl.kernel(body, out_type=jax.ShapeDtypeStruct((2, R, D), jnp.float32),
                     mesh=_SC_MESH, scratch_types=scratch,
                     compiler_params=pltpu.CompilerParams(
                         use_tc_tiling_on_sc=tc_tiling))


def _make_sc_deg():
    scratch = [
        pltpu.VMEM((CHUNKS_PER_W, C), jnp.int32),
        pltpu.VMEM((C, D_H), jnp.float32),
        pltpu.VMEM_SHARED((R, D_H), jnp.float32),
    ]

    def body(dstp, zrows, ones_h, deg_hbm, dst_v, ones_v, deg_sh):
        c = lax.axis_index("c")
        s = lax.axis_index("s")
        w = s * 2 + c
        base_r = s * ROWS_PER_SUB

        pltpu.sync_copy(dstp.at[pl.ds(w * CHUNKS_PER_W, CHUNKS_PER_W)], dst_v)

        @pl.loop(0, ROWS_PER_SUB // 128)
        def _(i):
            pltpu.sync_copy(zrows, deg_sh.at[pl.ds(base_r + i * 128, 128)])
        pltpu.sync_copy(ones_h, ones_v)

        plsc.subcore_barrier()

        @pl.loop(0, CHUNKS_PER_W)
        def _(k):
            pltpu.sync_copy(ones_v, deg_sh.at[dst_v.at[k]], add=True)

        plsc.subcore_barrier()

        pltpu.sync_copy(deg_sh.at[pl.ds(base_r, ROWS_PER_SUB)],
                        deg_hbm.at[c, pl.ds(base_r, ROWS_PER_SUB)])

    return pl.kernel(
        body, out_type=jax.ShapeDtypeStruct((2, R, D_H), jnp.float32),
        mesh=_SC_MESH, scratch_types=scratch)


_sc_agg_128 = _make_sc_agg(D_H)
_sc_deg = _make_sc_deg()


def _tc_pre(x_ref, w_ref, o_ref):
    o_ref[...] = jnp.dot(x_ref[...], w_ref[...],
                         preferred_element_type=jnp.float32)


def _tc_mid(aggp, degp, h_prev, Wr, b, g, be, Wln, h_out, t_out):
    deg = jnp.maximum(degp[0] + degp[1], 1.0)
    agg = aggp[0] + aggp[1]
    mean = agg / deg[:, None]
    z = mean + jnp.dot(h_prev[...], Wr[...],
                       preferred_element_type=jnp.float32) + b[...]
    m = jnp.mean(z, axis=0)
    v = jnp.mean((z - m) ** 2, axis=0)
    h = jnp.maximum((z - m) / jnp.sqrt(v + EPSV) * g[...] + be[...], 0.0)
    h_out[...] = h
    t_out[...] = jnp.dot(h, Wln[...], preferred_element_type=jnp.float32)


def _tc_fin(aggp, degp, h_prev, Wr, b, o_ref):
    deg = jnp.maximum(degp[0] + degp[1], 1.0)
    agg = aggp[0] + aggp[1]
    o_ref[...] = agg / deg[:, None] + jnp.dot(
        h_prev[...], Wr[...], preferred_element_type=jnp.float32) + b[...]


def kernel(x, edge_index, Wl1, Wr1, b1, g1, be1, Wl2, Wr2, b2, g2, be2,
           Wl3, Wr3, b3):
    src = edge_index[0]
    dst = edge_index[1]
    pad = E_PAD - E
    srcp = jnp.concatenate([src, jnp.zeros((pad,), jnp.int32)])
    srcp = srcp.reshape(E_PAD // C, C)
    dstp = jnp.concatenate([dst, jnp.full((pad,), N, jnp.int32)])
    dstp = dstp.reshape(E_PAD // C, C)
    z128 = jnp.zeros((128, D_H), jnp.float32)
    ones128 = jnp.ones((C, D_H), jnp.float32)

    f32 = jnp.float32
    t1 = pl.pallas_call(
        _tc_pre, out_shape=jax.ShapeDtypeStruct((N, D_H), f32))(x, Wl1)

    degp = _sc_deg(dstp, z128, ones128)
    degv = degp[:, :N, 0]
    agg1p = _sc_agg_128(t1, srcp, dstp, z128)

    h1, t2 = pl.pallas_call(
        _tc_mid,
        out_shape=[jax.ShapeDtypeStruct((N, D_H), f32),
                   jax.ShapeDtypeStruct((N, D_H), f32)],
    )(agg1p[:, :N], degv, x, Wr1, b1, g1, be1, Wl2)

    agg2p = _sc_agg_128(t2, srcp, dstp, z128)

    Wl3p = jnp.pad(Wl3, ((0, 0), (0, D_H - D_OUT)))
    h2, t3 = pl.pallas_call(
        _tc_mid,
        out_shape=[jax.ShapeDtypeStruct((N, D_H), f32),
                   jax.ShapeDtypeStruct((N, D_H), f32)],
    )(agg2p[:, :N], degv, h1, Wr2, b2, g2, be2, Wl3p)

    agg3p = _sc_agg_128(t3, srcp, dstp, z128)

    out = pl.pallas_call(
        _tc_fin, out_shape=jax.ShapeDtypeStruct((N, D_OUT), f32),
    )(agg3p[:, :N, :D_OUT], degv, h2, Wr3, b3)
    return out
